# baseline (speedup 1.0000x reference)
import functools
import math

import jax
import jax.numpy as jnp
from jax.experimental import pallas as pl
from jax.experimental.pallas import tpu as pltpu

NODE_DIM = 128
NUM_BASIS = 20
CUTOFF = 5.0
VOCAB = 100
PI = math.pi


def _edge_body(vx_ref, vy_ref, vz_ref, rbf_ref, fcut_ref, uvec_ref):
    vx = vx_ref[...]
    vy = vy_ref[...]
    vz = vz_ref[...]
    d2 = vx * vx + vy * vy + vz * vz
    inv = jax.lax.rsqrt(d2)
    d = d2 * inv
    x = (PI / CUTOFF) * d
    s1 = jnp.sin(x)
    c1 = jnp.cos(x)
    two_c = 2.0 * c1
    scale = math.sqrt(2.0 / CUTOFF) * inv
    terms = [s1 * scale]
    s_prev, s_cur = jnp.zeros_like(s1), s1
    for _ in range(NUM_BASIS - 1):
        s_prev, s_cur = s_cur, two_c * s_cur - s_prev
        terms.append(s_cur * scale)
    S = jnp.stack(terms, axis=1)
    rbf_ref[...] = jnp.transpose(S, (0, 2, 1))
    fc = 0.5 * (c1 + 1.0) * (d < CUTOFF).astype(jnp.float32)
    fcut_ref[...] = fc
    U = jnp.stack([vx * inv, vy * inv, vz * inv], axis=1)
    uvec_ref[...] = jnp.transpose(U, (0, 2, 1))


def _embed_body(ids_ref, tab_ref, out_ref):
    ids = ids_ref[...]
    lanes = jax.lax.broadcasted_iota(jnp.int32, (ids.shape[0], 128), 1)
    oh = (ids == lanes).astype(jnp.float32)
    out_ref[...] = jnp.dot(oh, tab_ref[...], preferred_element_type=jnp.float32)


@functools.partial(jax.jit, static_argnames=("interpret",))
def kernel(at_no, pos, edge_index, embed_table, interpret=False):
    E = edge_index.shape[1]
    N = at_no.shape[0]
    src, dst = edge_index[0], edge_index[1]
    px, py, pz = pos[:, 0], pos[:, 1], pos[:, 2]
    vx = jnp.take(px, dst) - jnp.take(px, src)
    vy = jnp.take(py, dst) - jnp.take(py, src)
    vz = jnp.take(pz, dst) - jnp.take(pz, src)
    ROWS = E // 128
    R = 40
    grid = ROWS // R
    vx2 = vx.reshape(ROWS, 128)
    vy2 = vy.reshape(ROWS, 128)
    vz2 = vz.reshape(ROWS, 128)
    rbf3, fcut2, uvec3 = pl.pallas_call(
        _edge_body,
        grid=(grid,),
        in_specs=[pl.BlockSpec((R, 128), lambda i: (i, 0))] * 3,
        out_specs=[
            pl.BlockSpec((R, 128, NUM_BASIS), lambda i: (i, 0, 0)),
            pl.BlockSpec((R, 128), lambda i: (i, 0)),
            pl.BlockSpec((R, 128, 3), lambda i: (i, 0, 0)),
        ],
        out_shape=[
            jax.ShapeDtypeStruct((ROWS, 128, NUM_BASIS), jnp.float32),
            jax.ShapeDtypeStruct((ROWS, 128), jnp.float32),
            jax.ShapeDtypeStruct((ROWS, 128, 3), jnp.float32),
        ],
        interpret=interpret,
    )(vx2, vy2, vz2)

    Bn = 2000
    tab = jnp.pad(embed_table, ((0, 128 - VOCAB), (0, 0)))
    ids2 = at_no.reshape(N, 1)
    x_scalar = pl.pallas_call(
        _embed_body,
        grid=(N // Bn,),
        in_specs=[
            pl.BlockSpec((Bn, 1), lambda i: (i, 0)),
            pl.BlockSpec((128, NODE_DIM), lambda i: (0, 0)),
        ],
        out_specs=pl.BlockSpec((Bn, NODE_DIM), lambda i: (i, 0)),
        out_shape=jax.ShapeDtypeStruct((N, NODE_DIM), jnp.float32),
        interpret=interpret,
    )(ids2, tab)

    return (
        x_scalar,
        rbf3.reshape(E, NUM_BASIS),
        fcut2.reshape(E, 1),
        uvec3.reshape(E, 3),
    )
